# baseline (speedup 1.0000x reference)
import functools

import jax
import jax.numpy as jnp
from jax import lax
from jax.experimental import pallas as pl
from jax.experimental.pallas import tpu as pltpu
from jax.experimental.pallas import tpu_sc as plsc

N_NODES = 10000
N_EDGES = 320000
D = 128

NP = 10112
ROWS_PER_TILE = NP // 16
N_TILES = 32
CHUNK = 128
CHUNKS_PER_TILE = 80
EP = N_TILES * CHUNKS_PER_TILE * CHUNK
K0 = 160
WINDOW = 40
PAD_DST = 10008


def _sc_deg_body(dst_hbm, ones_hbm, zrow_hbm, deg_out, dst_v, ones_v, deg_sh):
    c = lax.axis_index("c")
    s = lax.axis_index("s")
    g = c * 16 + s
    pltpu.sync_copy(dst_hbm.at[g], dst_v)
    pltpu.sync_copy(ones_hbm, ones_v)
    pltpu.sync_copy(zrow_hbm, deg_sh.at[pl.ds(s * ROWS_PER_TILE, ROWS_PER_TILE)])
    plsc.subcore_barrier()

    def body(j, carry):
        pltpu.sync_copy(ones_v, deg_sh.at[dst_v.at[j]], add=True)
        return carry

    lax.fori_loop(0, CHUNKS_PER_TILE, body, 0)
    plsc.subcore_barrier()
    sl = pl.ds(s * ROWS_PER_TILE, ROWS_PER_TILE)
    pltpu.sync_copy(deg_sh.at[sl], deg_out.at[c, sl])


def _sc_agg_body(hs_hbm, src_hbm, dst_hbm, zrow_hbm, out_hbm,
                 src_v, dst_v, stage_v, agg_sh, sem_a, sem_b):
    c = lax.axis_index("c")
    s = lax.axis_index("s")

    @pl.when(c == 0)
    def _():
        pltpu.sync_copy(zrow_hbm,
                        agg_sh.at[pl.ds(s * ROWS_PER_TILE, ROWS_PER_TILE)])
        plsc.subcore_barrier()

        st_a = stage_v.at[0]
        st_b = stage_v.at[1]
        n_pairs = WINDOW // 2

        def run_window(w):
            pltpu.sync_copy(src_hbm.at[s, pl.ds(w * WINDOW, WINDOW)],
                            src_v.at[pl.ds(0, WINDOW)])
            pltpu.sync_copy(dst_hbm.at[s, pl.ds(w * WINDOW, WINDOW)],
                            dst_v.at[pl.ds(0, WINDOW)])
            pltpu.async_copy(hs_hbm.at[src_v.at[0]], st_a, sem_a)

            def pair(j, carry):
                c0 = 2 * j
                pltpu.make_async_copy(hs_hbm.at[src_v.at[c0]], st_a, sem_a).wait()
                pltpu.async_copy(hs_hbm.at[src_v.at[c0 + 1]], st_b, sem_b)
                pltpu.sync_copy(st_a, agg_sh.at[dst_v.at[c0]], add=True)
                pltpu.make_async_copy(hs_hbm.at[src_v.at[c0 + 1]], st_b,
                                      sem_b).wait()

                @pl.when(j < n_pairs - 1)
                def _():
                    pltpu.async_copy(hs_hbm.at[src_v.at[c0 + 2]], st_a, sem_a)

                pltpu.sync_copy(st_b, agg_sh.at[dst_v.at[c0 + 1]], add=True)
                return carry

            lax.fori_loop(0, n_pairs, pair, 0)

        for w in range(K0 // WINDOW):
            run_window(w)

        plsc.subcore_barrier()
        sl = pl.ds(s * ROWS_PER_TILE, ROWS_PER_TILE)
        pltpu.sync_copy(agg_sh.at[sl], out_hbm.at[sl])


@functools.cache
def _build_sc():
    mesh = plsc.VectorSubcoreMesh(core_axis_name="c", subcore_axis_name="s")
    sc_deg = pl.kernel(
        _sc_deg_body,
        out_type=jax.ShapeDtypeStruct((2, NP, D), jnp.float32),
        mesh=mesh,
        scratch_types=[
            pltpu.VMEM((CHUNKS_PER_TILE, CHUNK), jnp.int32),
            pltpu.VMEM((CHUNK, D), jnp.float32),
            pltpu.VMEM_SHARED((NP, D), jnp.float32),
        ],
    )
    sc_agg = pl.kernel(
        _sc_agg_body,
        out_type=jax.ShapeDtypeStruct((NP, D), jnp.float32),
        mesh=mesh,
        scratch_types=[
            pltpu.VMEM((WINDOW + 8, CHUNK), jnp.int32),
            pltpu.VMEM((WINDOW + 8, CHUNK), jnp.int32),
            pltpu.VMEM((2, CHUNK, D), jnp.float32),
            pltpu.VMEM_SHARED((NP, D), jnp.float32),
            pltpu.SemaphoreType.DMA,
            pltpu.SemaphoreType.DMA,
        ],
    )
    return sc_deg, sc_agg


def _artanh(z):
    return 0.5 * jnp.log((1.0 + z) / (1.0 - z))


def _deg_inv(degp):
    deg = degp[0][:, 0:1] + degp[1][:, 0:1]
    return lax.rsqrt(jnp.maximum(deg, 1.0))


def _logmap0_scale(sq_norm):
    r = jnp.sqrt(jnp.maximum(sq_norm, 0.0))
    rc = jnp.clip(r, 1e-7, 1.0 - 1e-5)
    return _artanh(rc) / rc


def _expmap0(a):
    r = jnp.sqrt(jnp.sum(a * a, axis=1, keepdims=True))
    r = jnp.maximum(r, 1e-7)
    return jnp.tanh(r) * a / r


def _dense1_body(x_ref, w_ref, b_ref, degp_ref, out_ref):
    x = x_ref[...]
    y = x / (1.0 + x[:, 0:1])
    s2 = jnp.sum(y * y, axis=1, keepdims=True) - y[:, 0:1] ** 2
    sc = _logmap0_scale(s2)
    m = jnp.dot(y, w_ref[...], preferred_element_type=jnp.float32)
    h = sc * m + b_ref[...][None, :]
    out_ref[...] = h * _deg_inv(degp_ref[...])


def _dense2_body(agg_ref, degp_ref, w_ref, b_ref, out_ref):
    dinv = _deg_inv(degp_ref[...])
    agg = agg_ref[...] * dinv
    a = jnp.maximum(agg, 0.0)
    u = _expmap0(a)
    sc = _logmap0_scale(jnp.sum(u * u, axis=1, keepdims=True))
    v = sc * u
    h = jnp.dot(v, w_ref[...], preferred_element_type=jnp.float32)
    out_ref[...] = (h + b_ref[...][None, :]) * dinv


def _dense3_body(agg_ref, degp_ref, out_ref):
    dinv = _deg_inv(degp_ref[...])
    agg = agg_ref[...] * dinv
    out_ref[...] = _expmap0(jnp.maximum(agg, 0.0))


_out_nd = jax.ShapeDtypeStruct((NP, D), jnp.float32)
_dense1 = pl.pallas_call(_dense1_body, out_shape=_out_nd)
_dense2 = pl.pallas_call(_dense2_body, out_shape=_out_nd)
_dense3 = pl.pallas_call(_dense3_body, out_shape=_out_nd)


def kernel(x, edge_index, W1, b1, W2, b2):
    xp = jnp.pad(x, ((0, NP - N_NODES), (0, 0)))
    pad = EP - N_EDGES
    srcf = jnp.pad(edge_index[0], (0, pad))
    dstf = jnp.pad(edge_index[1], (0, pad), constant_values=PAD_DST)
    dst_deg = dstf.reshape(N_TILES, CHUNKS_PER_TILE, CHUNK)
    src = srcf.reshape(16, K0, CHUNK)
    dst = dstf.reshape(16, K0, CHUNK)

    w1s = jnp.concatenate([jnp.zeros((1, D), jnp.float32), W1], axis=0)
    ones = jnp.ones((CHUNK, D), jnp.float32)
    zrow = jnp.zeros((ROWS_PER_TILE, D), jnp.float32)

    sc_deg, sc_agg = _build_sc()
    degp = sc_deg(dst_deg, ones, zrow)
    hs1 = _dense1(xp, w1s, b1, degp)
    agg1 = sc_agg(hs1, src, dst, zrow)
    hs2 = _dense2(agg1, degp, W2, b2)
    agg2 = sc_agg(hs2, src, dst, zrow)
    out = _dense3(agg2, degp)
    return out[:N_NODES]


# speedup vs baseline: 1.2182x; 1.2182x over previous
import functools

import jax
import jax.numpy as jnp
from jax import lax
from jax.experimental import pallas as pl
from jax.experimental.pallas import tpu as pltpu
from jax.experimental.pallas import tpu_sc as plsc

N_NODES = 10000
N_EDGES = 320000
D = 128

NP = 10112
ROWS_PER_TILE = NP // 16
N_TILES = 32
CHUNK = 128
CHUNKS_PER_TILE = 80
EP = N_TILES * CHUNKS_PER_TILE * CHUNK
K0 = 80
WINDOW = 40
PAD_DST = 10008


def _sc_deg_body(dst_hbm, ones_hbm, zrow_hbm, deg_out, dst_v, ones_v, deg_sh):
    c = lax.axis_index("c")
    s = lax.axis_index("s")
    g = c * 16 + s
    pltpu.sync_copy(dst_hbm.at[g], dst_v)
    pltpu.sync_copy(ones_hbm, ones_v)
    pltpu.sync_copy(zrow_hbm, deg_sh.at[pl.ds(s * ROWS_PER_TILE, ROWS_PER_TILE)])
    plsc.subcore_barrier()

    def body(j, carry):
        pltpu.sync_copy(ones_v, deg_sh.at[dst_v.at[j]], add=True)
        return carry

    lax.fori_loop(0, CHUNKS_PER_TILE, body, 0)
    plsc.subcore_barrier()
    sl = pl.ds(s * ROWS_PER_TILE, ROWS_PER_TILE)
    pltpu.sync_copy(deg_sh.at[sl], deg_out.at[c, sl])


def _sc_agg_body(hs_hbm, src_hbm, dst_hbm, zrow_hbm, out_hbm,
                 src_v, dst_v, stage_v, agg_sh, sem_a, sem_b):
    c = lax.axis_index("c")
    s = lax.axis_index("s")
    g = c * 16 + s
    pltpu.sync_copy(zrow_hbm, agg_sh.at[pl.ds(s * ROWS_PER_TILE, ROWS_PER_TILE)])
    plsc.subcore_barrier()

    st_a = stage_v.at[0]
    st_b = stage_v.at[1]
    n_pairs = WINDOW // 2

    def run_window(w):
        pltpu.sync_copy(src_hbm.at[g, pl.ds(w * WINDOW, WINDOW)],
                        src_v.at[pl.ds(0, WINDOW)])
        pltpu.sync_copy(dst_hbm.at[g, pl.ds(w * WINDOW, WINDOW)],
                        dst_v.at[pl.ds(0, WINDOW)])
        pltpu.async_copy(hs_hbm.at[src_v.at[0]], st_a, sem_a)

        def pair(j, carry):
            c0 = 2 * j
            pltpu.make_async_copy(hs_hbm.at[src_v.at[c0]], st_a, sem_a).wait()
            pltpu.async_copy(hs_hbm.at[src_v.at[c0 + 1]], st_b, sem_b)
            pltpu.sync_copy(st_a, agg_sh.at[dst_v.at[c0]], add=True)
            pltpu.make_async_copy(hs_hbm.at[src_v.at[c0 + 1]], st_b,
                                  sem_b).wait()

            @pl.when(j < n_pairs - 1)
            def _():
                pltpu.async_copy(hs_hbm.at[src_v.at[c0 + 2]], st_a, sem_a)

            pltpu.sync_copy(st_b, agg_sh.at[dst_v.at[c0 + 1]], add=True)
            return carry

        lax.fori_loop(0, n_pairs, pair, 0)

    for w in range(K0 // WINDOW):
        run_window(w)

    plsc.subcore_barrier()
    sl = pl.ds(s * ROWS_PER_TILE, ROWS_PER_TILE)
    pltpu.sync_copy(agg_sh.at[sl], out_hbm.at[c, sl])


@functools.cache
def _build_sc():
    mesh = plsc.VectorSubcoreMesh(core_axis_name="c", subcore_axis_name="s")
    sc_deg = pl.kernel(
        _sc_deg_body,
        out_type=jax.ShapeDtypeStruct((2, NP, D), jnp.float32),
        mesh=mesh,
        scratch_types=[
            pltpu.VMEM((CHUNKS_PER_TILE, CHUNK), jnp.int32),
            pltpu.VMEM((CHUNK, D), jnp.float32),
            pltpu.VMEM_SHARED((NP, D), jnp.float32),
        ],
    )
    sc_agg = pl.kernel(
        _sc_agg_body,
        out_type=jax.ShapeDtypeStruct((2, NP, D), jnp.float32),
        mesh=mesh,
        scratch_types=[
            pltpu.VMEM((WINDOW + 8, CHUNK), jnp.int32),
            pltpu.VMEM((WINDOW + 8, CHUNK), jnp.int32),
            pltpu.VMEM((2, CHUNK, D), jnp.float32),
            pltpu.VMEM_SHARED((NP, D), jnp.float32),
            pltpu.SemaphoreType.DMA,
            pltpu.SemaphoreType.DMA,
        ],
    )
    return sc_deg, sc_agg


def _artanh(z):
    return 0.5 * jnp.log((1.0 + z) / (1.0 - z))


def _deg_inv(degp):
    deg = degp[0][:, 0:1] + degp[1][:, 0:1]
    return lax.rsqrt(jnp.maximum(deg, 1.0))


def _logmap0_scale(sq_norm):
    r = jnp.sqrt(jnp.maximum(sq_norm, 0.0))
    rc = jnp.clip(r, 1e-7, 1.0 - 1e-5)
    return _artanh(rc) / rc


def _expmap0(a):
    r = jnp.sqrt(jnp.sum(a * a, axis=1, keepdims=True))
    r = jnp.maximum(r, 1e-7)
    return jnp.tanh(r) * a / r


def _dense1_body(x_ref, w_ref, b_ref, degp_ref, out_ref):
    x = x_ref[...]
    y = x / (1.0 + x[:, 0:1])
    s2 = jnp.sum(y * y, axis=1, keepdims=True) - y[:, 0:1] ** 2
    sc = _logmap0_scale(s2)
    m = jnp.dot(y, w_ref[...], preferred_element_type=jnp.float32)
    h = sc * m + b_ref[...][None, :]
    out_ref[...] = h * _deg_inv(degp_ref[...])


def _dense2_body(aggp_ref, degp_ref, w_ref, b_ref, out_ref):
    dinv = _deg_inv(degp_ref[...])
    agg = (aggp_ref[0] + aggp_ref[1]) * dinv
    a = jnp.maximum(agg, 0.0)
    u = _expmap0(a)
    sc = _logmap0_scale(jnp.sum(u * u, axis=1, keepdims=True))
    v = sc * u
    h = jnp.dot(v, w_ref[...], preferred_element_type=jnp.float32)
    out_ref[...] = (h + b_ref[...][None, :]) * dinv


def _dense3_body(aggp_ref, degp_ref, out_ref):
    dinv = _deg_inv(degp_ref[...])
    agg = (aggp_ref[0] + aggp_ref[1]) * dinv
    out_ref[...] = _expmap0(jnp.maximum(agg, 0.0))


_out_nd = jax.ShapeDtypeStruct((NP, D), jnp.float32)
_dense1 = pl.pallas_call(_dense1_body, out_shape=_out_nd)
_dense2 = pl.pallas_call(_dense2_body, out_shape=_out_nd)
_dense3 = pl.pallas_call(_dense3_body, out_shape=_out_nd)


def kernel(x, edge_index, W1, b1, W2, b2):
    xp = jnp.pad(x, ((0, NP - N_NODES), (0, 0)))
    pad = EP - N_EDGES
    srcf = jnp.pad(edge_index[0], (0, pad))
    dstf = jnp.pad(edge_index[1], (0, pad), constant_values=PAD_DST)
    dst_deg = dstf.reshape(N_TILES, CHUNKS_PER_TILE, CHUNK)
    src = srcf.reshape(N_TILES, K0, CHUNK)
    dst = dstf.reshape(N_TILES, K0, CHUNK)

    w1s = jnp.concatenate([jnp.zeros((1, D), jnp.float32), W1], axis=0)
    ones = jnp.ones((CHUNK, D), jnp.float32)
    zrow = jnp.zeros((ROWS_PER_TILE, D), jnp.float32)

    sc_deg, sc_agg = _build_sc()
    degp = sc_deg(dst_deg, ones, zrow)
    hs1 = _dense1(xp, w1s, b1, degp)
    agg1 = sc_agg(hs1, src, dst, zrow)
    hs2 = _dense2(agg1, degp, W2, b2)
    agg2 = sc_agg(hs2, src, dst, zrow)
    out = _dense3(agg2, degp)
    return out[:N_NODES]


# speedup vs baseline: 1.2250x; 1.0057x over previous
import functools

import jax
import jax.numpy as jnp
from jax import lax
from jax.experimental import pallas as pl
from jax.experimental.pallas import tpu as pltpu
from jax.experimental.pallas import tpu_sc as plsc

N_NODES = 10000
N_EDGES = 320000
D = 128

NP = 10112
ROWS_PER_TILE = NP // 16
N_TILES = 32
CHUNK = 128
CHUNKS_PER_TILE = 80
EP = N_TILES * CHUNKS_PER_TILE * CHUNK
K0 = 80
WINDOW = 40
PAD_DST = 10008


def _sc_deg_body(dst_hbm, ones_hbm, zrow_hbm, deg_out, dst_v, ones_v, deg_sh):
    c = lax.axis_index("c")
    s = lax.axis_index("s")
    g = c * 16 + s
    pltpu.sync_copy(dst_hbm.at[g], dst_v)
    pltpu.sync_copy(ones_hbm, ones_v)
    pltpu.sync_copy(zrow_hbm, deg_sh.at[pl.ds(s * ROWS_PER_TILE, ROWS_PER_TILE)])
    plsc.subcore_barrier()

    def body(j, carry):
        pltpu.sync_copy(ones_v, deg_sh.at[dst_v.at[j]], add=True)
        return carry

    lax.fori_loop(0, CHUNKS_PER_TILE, body, 0)
    plsc.subcore_barrier()
    sl = pl.ds(s * ROWS_PER_TILE, ROWS_PER_TILE)
    pltpu.sync_copy(deg_sh.at[sl], deg_out.at[c, sl])


def _sc_agg_body(hs_hbm, src_hbm, dst_hbm, zrow_hbm, out_hbm,
                 src_v, dst_v, stage_v, agg_sh, sem_a, sem_b, sem_sa, sem_sb):
    c = lax.axis_index("c")
    s = lax.axis_index("s")
    g = c * 16 + s
    pltpu.sync_copy(zrow_hbm, agg_sh.at[pl.ds(s * ROWS_PER_TILE, ROWS_PER_TILE)])
    plsc.subcore_barrier()

    st_a = stage_v.at[0]
    st_b = stage_v.at[1]
    n_pairs = WINDOW // 2

    def run_window(w):
        pltpu.sync_copy(src_hbm.at[g, pl.ds(w * WINDOW, WINDOW)],
                        src_v.at[pl.ds(0, WINDOW)])
        pltpu.sync_copy(dst_hbm.at[g, pl.ds(w * WINDOW, WINDOW)],
                        dst_v.at[pl.ds(0, WINDOW)])
        pltpu.async_copy(hs_hbm.at[src_v.at[0]], st_a, sem_a)
        pltpu.async_copy(hs_hbm.at[src_v.at[1]], st_b, sem_b)

        def pair(j, carry):
            c0 = 2 * j
            pltpu.make_async_copy(hs_hbm.at[src_v.at[c0]], st_a, sem_a).wait()
            pltpu.async_copy(st_a, agg_sh.at[dst_v.at[c0]], sem_sa, add=True)
            pltpu.make_async_copy(hs_hbm.at[src_v.at[c0 + 1]], st_b,
                                  sem_b).wait()
            pltpu.async_copy(st_b, agg_sh.at[dst_v.at[c0 + 1]], sem_sb,
                             add=True)
            pltpu.make_async_copy(st_a, agg_sh.at[dst_v.at[c0]], sem_sa).wait()

            @pl.when(j < n_pairs - 1)
            def _():
                pltpu.async_copy(hs_hbm.at[src_v.at[c0 + 2]], st_a, sem_a)

            pltpu.make_async_copy(st_b, agg_sh.at[dst_v.at[c0 + 1]],
                                  sem_sb).wait()

            @pl.when(j < n_pairs - 1)
            def _():
                pltpu.async_copy(hs_hbm.at[src_v.at[c0 + 3]], st_b, sem_b)

            return carry

        lax.fori_loop(0, n_pairs, pair, 0)

    for w in range(K0 // WINDOW):
        run_window(w)

    plsc.subcore_barrier()
    sl = pl.ds(s * ROWS_PER_TILE, ROWS_PER_TILE)
    pltpu.sync_copy(agg_sh.at[sl], out_hbm.at[c, sl])


@functools.cache
def _build_sc():
    mesh = plsc.VectorSubcoreMesh(core_axis_name="c", subcore_axis_name="s")
    sc_deg = pl.kernel(
        _sc_deg_body,
        out_type=jax.ShapeDtypeStruct((2, NP, D), jnp.float32),
        mesh=mesh,
        scratch_types=[
            pltpu.VMEM((CHUNKS_PER_TILE, CHUNK), jnp.int32),
            pltpu.VMEM((CHUNK, D), jnp.float32),
            pltpu.VMEM_SHARED((NP, D), jnp.float32),
        ],
    )
    sc_agg = pl.kernel(
        _sc_agg_body,
        out_type=jax.ShapeDtypeStruct((2, NP, D), jnp.float32),
        mesh=mesh,
        scratch_types=[
            pltpu.VMEM((WINDOW + 8, CHUNK), jnp.int32),
            pltpu.VMEM((WINDOW + 8, CHUNK), jnp.int32),
            pltpu.VMEM((2, CHUNK, D), jnp.float32),
            pltpu.VMEM_SHARED((NP, D), jnp.float32),
            pltpu.SemaphoreType.DMA,
            pltpu.SemaphoreType.DMA,
            pltpu.SemaphoreType.DMA,
            pltpu.SemaphoreType.DMA,
        ],
    )
    return sc_deg, sc_agg


def _artanh(z):
    return 0.5 * jnp.log((1.0 + z) / (1.0 - z))


def _deg_inv(degp):
    deg = degp[0][:, 0:1] + degp[1][:, 0:1]
    return lax.rsqrt(jnp.maximum(deg, 1.0))


def _logmap0_scale(sq_norm):
    r = jnp.sqrt(jnp.maximum(sq_norm, 0.0))
    rc = jnp.clip(r, 1e-7, 1.0 - 1e-5)
    return _artanh(rc) / rc


def _expmap0(a):
    r = jnp.sqrt(jnp.sum(a * a, axis=1, keepdims=True))
    r = jnp.maximum(r, 1e-7)
    return jnp.tanh(r) * a / r


def _dense1_body(x_ref, w_ref, b_ref, degp_ref, out_ref):
    x = x_ref[...]
    y = x / (1.0 + x[:, 0:1])
    s2 = jnp.sum(y * y, axis=1, keepdims=True) - y[:, 0:1] ** 2
    sc = _logmap0_scale(s2)
    m = jnp.dot(y, w_ref[...], preferred_element_type=jnp.float32)
    h = sc * m + b_ref[...][None, :]
    out_ref[...] = h * _deg_inv(degp_ref[...])


def _dense2_body(aggp_ref, degp_ref, w_ref, b_ref, out_ref):
    dinv = _deg_inv(degp_ref[...])
    agg = (aggp_ref[0] + aggp_ref[1]) * dinv
    a = jnp.maximum(agg, 0.0)
    u = _expmap0(a)
    sc = _logmap0_scale(jnp.sum(u * u, axis=1, keepdims=True))
    v = sc * u
    h = jnp.dot(v, w_ref[...], preferred_element_type=jnp.float32)
    out_ref[...] = (h + b_ref[...][None, :]) * dinv


def _dense3_body(aggp_ref, degp_ref, out_ref):
    dinv = _deg_inv(degp_ref[...])
    agg = (aggp_ref[0] + aggp_ref[1]) * dinv
    out_ref[...] = _expmap0(jnp.maximum(agg, 0.0))


_out_nd = jax.ShapeDtypeStruct((NP, D), jnp.float32)
_dense1 = pl.pallas_call(_dense1_body, out_shape=_out_nd)
_dense2 = pl.pallas_call(_dense2_body, out_shape=_out_nd)
_dense3 = pl.pallas_call(_dense3_body, out_shape=_out_nd)


def kernel(x, edge_index, W1, b1, W2, b2):
    xp = jnp.pad(x, ((0, NP - N_NODES), (0, 0)))
    pad = EP - N_EDGES
    srcf = jnp.pad(edge_index[0], (0, pad))
    dstf = jnp.pad(edge_index[1], (0, pad), constant_values=PAD_DST)
    dst_deg = dstf.reshape(N_TILES, CHUNKS_PER_TILE, CHUNK)
    src = srcf.reshape(N_TILES, K0, CHUNK)
    dst = dstf.reshape(N_TILES, K0, CHUNK)

    w1s = jnp.concatenate([jnp.zeros((1, D), jnp.float32), W1], axis=0)
    ones = jnp.ones((CHUNK, D), jnp.float32)
    zrow = jnp.zeros((ROWS_PER_TILE, D), jnp.float32)

    sc_deg, sc_agg = _build_sc()
    degp = sc_deg(dst_deg, ones, zrow)
    hs1 = _dense1(xp, w1s, b1, degp)
    agg1 = sc_agg(hs1, src, dst, zrow)
    hs2 = _dense2(agg1, degp, W2, b2)
    agg2 = sc_agg(hs2, src, dst, zrow)
    out = _dense3(agg2, degp)
    return out[:N_NODES]
